# speedup vs baseline: 3.8060x; 3.8060x over previous
import functools

import jax
import jax.numpy as jnp
from jax import lax
from jax.experimental import pallas as pl
from jax.experimental.pallas import tpu as pltpu
from jax.experimental.pallas import tpu_sc as plsc

GAMMA = 0.1
EPS = 0.1
NUM_LAYERS = 4

_NC = 2
_NS = 16
_NW = _NC * _NS
_CHUNK = 128


def _dotT(a, b):
  return lax.dot_general(a, b, (((1,), (1,)), ((), ())),
                         preferred_element_type=jnp.float32)


def _dot(a, b):
  return lax.dot_general(a, b, (((1,), (0,)), ((), ())),
                         preferred_element_type=jnp.float32)


@functools.lru_cache(maxsize=None)
def _make_agg(NP, EP, H):
  ept = EP // _NW
  nchunks = ept // _CHUNK
  rpt = NP // _NS
  mesh = plsc.VectorSubcoreMesh(core_axis_name="c", subcore_axis_name="s",
                                num_cores=_NC, num_subcores=_NS)

  @functools.partial(
      pl.kernel,
      out_type=jax.ShapeDtypeStruct((_NC * NP, H), jnp.float32),
      mesh=mesh,
      scratch_types=[
          pltpu.VMEM((_CHUNK,), jnp.int32),
          pltpu.VMEM((1, _CHUNK), jnp.int32),
          pltpu.VMEM((_CHUNK, H), jnp.float32),
          pltpu.VMEM((_CHUNK, H), jnp.float32),
          pltpu.VMEM_SHARED((NP, H), jnp.float32),
          pltpu.SemaphoreType.DMA,
      ],
  )
  def agg(h_hbm, src_hbm, dst_hbm, out_hbm, src_v, dst_v, rows_v, zero_v,
          acc_sh, sem):
    c = lax.axis_index("c")
    s = lax.axis_index("s")
    wid = c * _NS + s

    def zrow(r, carry):
      for j in range(H // 16):
        zero_v[r, pl.ds(j * 16, 16)] = jnp.zeros((16,), jnp.float32)
      return carry
    lax.fori_loop(0, _CHUNK, zrow, 0)
    for j in range(rpt // _CHUNK):
      pltpu.sync_copy(zero_v, acc_sh.at[pl.ds(s * rpt + j * _CHUNK, _CHUNK)])
    plsc.subcore_barrier()

    base0 = wid * ept

    def chunk(i, carry):
      base = base0 + i * _CHUNK
      pltpu.sync_copy(src_hbm.at[pl.ds(base, _CHUNK)], src_v)
      pltpu.sync_copy(dst_hbm.at[pl.ds(base, _CHUNK)], dst_v.at[0])
      pltpu.async_copy(h_hbm.at[src_v], rows_v, sem).wait()
      pltpu.sync_copy(rows_v, acc_sh.at[dst_v.at[0]], add=True)
      return carry
    lax.fori_loop(0, nchunks, chunk, 0)

    plsc.subcore_barrier()
    pltpu.sync_copy(acc_sh.at[pl.ds(s * rpt, rpt)],
                    out_hbm.at[pl.ds(c * NP + s * rpt, rpt)])

  return agg


def _emb_body(x_ref, w_ref, b_ref, o_ref):
  o_ref[...] = _dotT(x_ref[...], w_ref[...]) + b_ref[...]


def _layer_body(h_ref, a0_ref, a1_ref, wa_ref, wlin_ref, b_ref, o_ref):
  h = h_ref[...]
  hA = _dotT(h, wa_ref[...]) - _dot(h, wa_ref[...]) - GAMMA * h
  neigh = _dotT(a0_ref[...] + a1_ref[...], wlin_ref[...])
  conv = hA + neigh + b_ref[...]
  o_ref[...] = h + EPS * jnp.tanh(conv)


def _readout_body(h_ref, w1_ref, b1_ref, w2_ref, b2_ref, o_ref):
  t = _dotT(h_ref[...], w1_ref[...]) + b1_ref[...]
  t = jnp.where(t > 0, t, 0.01 * t)
  t = _dotT(t, w2_ref[...]) + b2_ref[...]
  o_ref[...] = jnp.where(t > 0, t, 0.01 * t)


def kernel(x, edge_index, batch, W_emb, b_emb, W_A, bias_conv, W_lin,
           W_r1, b_r1, W_r2, b_r2):
  N, D = x.shape
  H = W_emb.shape[0]
  OUT = W_r2.shape[0]
  E = edge_index.shape[1]

  NP = -(-N // (_NS * _CHUNK)) * (_NS * _CHUNK)
  EP = -(-E // (_NW * _CHUNK)) * (_NW * _CHUNK)

  src = edge_index[0]
  dst = edge_index[1]
  if EP > E:
    src = jnp.concatenate([src, jnp.zeros((EP - E,), jnp.int32)])
    dst = jnp.concatenate([dst, jnp.full((EP - E,), NP - 1, jnp.int32)])
  x_p = jnp.pad(x, ((0, NP - N), (0, 0))) if NP > N else x

  agg_call = _make_agg(NP, EP, H)

  emb = pl.pallas_call(
      _emb_body, out_shape=jax.ShapeDtypeStruct((NP, H), jnp.float32))
  layer = pl.pallas_call(
      _layer_body, out_shape=jax.ShapeDtypeStruct((NP, H), jnp.float32))
  readout = pl.pallas_call(
      _readout_body, out_shape=jax.ShapeDtypeStruct((NP, OUT), jnp.float32))

  h = emb(x_p, W_emb, b_emb.reshape(1, H))
  for _ in range(NUM_LAYERS):
    parts = agg_call(h, src, dst)
    h = layer(h, parts[:NP], parts[NP:], W_A, W_lin, bias_conv.reshape(1, H))
  out = readout(h, W_r1, b_r1.reshape(1, -1), W_r2, b_r2.reshape(1, OUT))
  return out[:N]
